# baseline (speedup 1.0000x reference)
import functools

import jax
import jax.numpy as jnp
import numpy as np
from jax.experimental import pallas as pl
from jax.experimental.pallas import tpu as pltpu

B, S, D, H, DH = 1, 2048, 2048, 16, 128
VIS_START, VIS_END = 1, 513
INSTR_START, INSTR_END = 513, 1992
KEEP_V = 256
ROPE_THETA = 10000.0
NV = VIS_END - VIS_START
NQT = INSTR_END - INSTR_START
NQT_PAD = 1480
_SCALE = float(np.float32(1.0) / np.sqrt(np.float32(DH)))
_BIGPOS = np.int32(1 << 24)


def _rope_tables():
    inv_freq = 1.0 / (ROPE_THETA ** (jnp.arange(0, DH, 2, dtype=jnp.float32) / DH))
    pos = jnp.arange(S, dtype=jnp.float32)
    freqs = pos[:, None] * inv_freq[None, :]
    emb = jnp.concatenate([freqs, freqs], axis=-1)
    return jnp.cos(emb), jnp.sin(emb)


def _qkv_rope_kernel(x_ref, wq_ref, wk_ref, wv_ref, cos_ref, sin_ref,
                     sc_ref, qb_ref, kb_ref, vb_ref):
    xb = x_ref[...]
    cos = cos_ref[...]
    sinp = sin_ref[...]
    sinn = jnp.concatenate([-sinp[:, :DH // 2], sinp[:, DH // 2:]], axis=1)

    def rope(t2, i):
        t = t2[:, i * DH:(i + 1) * DH]
        r = pltpu.roll(t, DH // 2, 1)
        return t * cos + r * sinn

    q2 = jax.lax.dot(xb, wq_ref[...].astype(jnp.bfloat16),
                     preferred_element_type=jnp.float32)
    k2 = jax.lax.dot(xb, wk_ref[...].astype(jnp.bfloat16),
                     preferred_element_type=jnp.float32)
    for i in range(2):
        roped_q = rope(q2, i)
        roped_k = rope(k2, i)
        qb_ref[i] = (roped_q * _SCALE).astype(jnp.bfloat16)
        kb_ref[i] = roped_k.astype(jnp.bfloat16)
        qtb = roped_q[INSTR_START:INSTR_START + NQT_PAD].astype(jnp.bfloat16)
        kvb = roped_k[VIS_START:VIS_END].astype(jnp.bfloat16)
        sc = jax.lax.dot_general(
            qtb, kvb, (((1,), (1,)), ((), ())),
            preferred_element_type=jnp.float32)
        sc_ref[i] = sc[:NQT]
    v2 = jax.lax.dot(xb, wv_ref[...].astype(jnp.bfloat16),
                     preferred_element_type=jnp.float32)
    for i in range(2):
        vb_ref[i] = v2[:, i * DH:(i + 1) * DH].astype(jnp.bfloat16)


def _qkv_rope(xb, wq, wk, wv, cos, sin):
    outb = jax.ShapeDtypeStruct((H, S, DH), jnp.bfloat16)
    return pl.pallas_call(
        _qkv_rope_kernel,
        grid=(H // 2,),
        in_specs=[
            pl.BlockSpec((S, D), lambda h: (0, 0)),
            pl.BlockSpec((D, 2 * DH), lambda h: (0, h)),
            pl.BlockSpec((D, 2 * DH), lambda h: (0, h)),
            pl.BlockSpec((D, 2 * DH), lambda h: (0, h)),
            pl.BlockSpec((S, DH), lambda h: (0, 0)),
            pl.BlockSpec((S, DH), lambda h: (0, 0)),
        ],
        out_specs=[
            pl.BlockSpec((2, NQT, NV), lambda h: (h, 0, 0)),
            pl.BlockSpec((2, S, DH), lambda h: (h, 0, 0)),
            pl.BlockSpec((2, S, DH), lambda h: (h, 0, 0)),
            pl.BlockSpec((2, S, DH), lambda h: (h, 0, 0)),
        ],
        out_shape=[
            jax.ShapeDtypeStruct((H, NQT, NV), jnp.float32),
            outb, outb, outb,
        ],
    )(xb, wq, wk, wv, cos, sin)


def _select_kernel(imp_ref, keep_ref):
    imp = imp_ref[...]
    a = jnp.broadcast_to(imp, (NV, NV))
    at = jnp.transpose(a)
    ii = jax.lax.broadcasted_iota(jnp.int32, (NV, NV), 0)
    jj = jax.lax.broadcasted_iota(jnp.int32, (NV, NV), 1)
    beats = (at > a) | ((at == a) & (ii < jj))
    rank = jnp.sum(beats.astype(jnp.int32), axis=0, keepdims=True)
    keep_ref[...] = (rank < KEEP_V).astype(jnp.int32)


def _select(imp):
    return pl.pallas_call(
        _select_kernel,
        out_shape=jax.ShapeDtypeStruct((1, NV), jnp.int32),
    )(imp)


_BQ = 512


def _attn_kernel(q_ref, k_ref, v_ref, pos_ref, o_ref, mask_ref, *, iblk, kext):
    h = pl.program_id(0)

    @pl.when(h == 0)
    def _():
        qpos = iblk * _BQ + jax.lax.broadcasted_iota(jnp.int32, (_BQ, 1), 0)
        allowed = pos_ref[0:1, :] <= qpos
        mask_ref[...] = jnp.where(allowed, 0.0, -jnp.inf)

    s = jax.lax.dot_general(
        q_ref[0], k_ref[0], (((1,), (1,)), ((), ())),
        preferred_element_type=jnp.float32)
    e = jnp.exp(s + mask_ref[...])
    den = jnp.sum(e, axis=1, keepdims=True)
    w = (e * (1.0 / den)).astype(jnp.bfloat16)
    o_ref[...] = jax.lax.dot(w, v_ref[0], preferred_element_type=jnp.float32)


def _attention(qb, kb, vb, posval):
    o_parts = []
    for i in range(S // _BQ):
        kext = (i + 1) * _BQ
        o_parts.append(pl.pallas_call(
            functools.partial(_attn_kernel, iblk=i, kext=kext),
            grid=(H,),
            in_specs=[
                pl.BlockSpec((1, _BQ, DH), lambda h, i=i: (h, i, 0)),
                pl.BlockSpec((1, kext, DH), lambda h: (h, 0, 0)),
                pl.BlockSpec((1, kext, DH), lambda h: (h, 0, 0)),
                pl.BlockSpec((8, kext), lambda h: (0, 0)),
            ],
            out_specs=pl.BlockSpec((_BQ, DH), lambda h: (0, h)),
            out_shape=jax.ShapeDtypeStruct((_BQ, H * DH), jnp.float32),
            scratch_shapes=[pltpu.VMEM((_BQ, kext), jnp.float32)],
        )(qb, kb, vb, posval))
    return jnp.concatenate(o_parts, axis=0)


_BSO = 512


def _outproj_kernel(o_ref, wo_ref, out_ref):
    ob = o_ref[...].astype(jnp.bfloat16)
    out_ref[...] = jax.lax.dot(ob, wo_ref[...].astype(jnp.bfloat16),
                               preferred_element_type=jnp.float32)


def _outproj(o, wo):
    return pl.pallas_call(
        _outproj_kernel,
        grid=(S // _BSO,),
        in_specs=[
            pl.BlockSpec((_BSO, H * DH), lambda i: (i, 0)),
            pl.BlockSpec((H * DH, D), lambda i: (0, 0)),
        ],
        out_specs=pl.BlockSpec((_BSO, D), lambda i: (i, 0)),
        out_shape=jax.ShapeDtypeStruct((S, D), jnp.float32),
    )(o, wo)


def kernel(hidden_states, Wq, Wk, Wv, Wo):
    x = hidden_states[0]
    cos, sin = _rope_tables()
    xb = x.astype(jnp.bfloat16)

    sc, qb, kb, vb = _qkv_rope(xb, Wq, Wk, Wv, cos, sin)

    scores = sc[None] * _SCALE
    probs = jax.nn.softmax(scores.astype(jnp.float32), axis=-1)
    imp = probs.sum(axis=(0, 1, 2))
    keep = _select(imp[None])

    posidx = jnp.arange(S, dtype=jnp.int32)
    vis_pos = jnp.where(keep[0] == 1, posidx[VIS_START:VIS_END], _BIGPOS)
    posval = jnp.concatenate(
        [posidx[:VIS_START], vis_pos, posidx[VIS_END:]])
    posval = jnp.broadcast_to(posval[None, :], (8, S))

    o = _attention(qb, kb, vb, posval)
    out = _outproj(o, Wo)
    return out[None]


# speedup vs baseline: 1.0282x; 1.0282x over previous
import functools

import jax
import jax.numpy as jnp
import numpy as np
from jax.experimental import pallas as pl
from jax.experimental.pallas import tpu as pltpu

B, S, D, H, DH = 1, 2048, 2048, 16, 128
VIS_START, VIS_END = 1, 513
INSTR_START, INSTR_END = 513, 1992
KEEP_V = 256
ROPE_THETA = 10000.0
NV = VIS_END - VIS_START
NQT = INSTR_END - INSTR_START
NQT_PAD = 1480
_SCALE = float(np.float32(1.0) / np.sqrt(np.float32(DH)))
_BIGPOS = np.int32(1 << 24)


def _rope_tables():
    inv_freq = 1.0 / (ROPE_THETA ** (jnp.arange(0, DH, 2, dtype=jnp.float32) / DH))
    pos = jnp.arange(S, dtype=jnp.float32)
    freqs = pos[:, None] * inv_freq[None, :]
    emb = jnp.concatenate([freqs, freqs], axis=-1)
    return jnp.cos(emb), jnp.sin(emb)


def _qkv_rope_kernel(x_ref, wq_ref, wk_ref, wv_ref, cos_ref, sin_ref,
                     qt_ref, kv_ref, qb_ref, kb_ref, vb_ref):
    xb = x_ref[...]
    cos = cos_ref[...]
    sinp = sin_ref[...]
    sinn = jnp.concatenate([-sinp[:, :DH // 2], sinp[:, DH // 2:]], axis=1)

    def rope(t2, i):
        t = t2[:, i * DH:(i + 1) * DH]
        r = pltpu.roll(t, DH // 2, 1)
        return t * cos + r * sinn

    q2 = jax.lax.dot(xb, wq_ref[...].astype(jnp.bfloat16),
                     preferred_element_type=jnp.float32)
    for i in range(2):
        roped = rope(q2, i)
        qt_ref[i] = roped[INSTR_START:INSTR_START + NQT_PAD]
        qb_ref[i] = (roped * _SCALE).astype(jnp.bfloat16)
    k2 = jax.lax.dot(xb, wk_ref[...].astype(jnp.bfloat16),
                     preferred_element_type=jnp.float32)
    for i in range(2):
        roped = rope(k2, i)
        kv_ref[i] = roped[VIS_START:VIS_END]
        kb_ref[i] = roped.astype(jnp.bfloat16)
    v2 = jax.lax.dot(xb, wv_ref[...].astype(jnp.bfloat16),
                     preferred_element_type=jnp.float32)
    for i in range(2):
        vb_ref[i] = v2[:, i * DH:(i + 1) * DH].astype(jnp.bfloat16)


def _qkv_rope(xb, wq, wk, wv, cos, sin):
    outb = jax.ShapeDtypeStruct((H, S, DH), jnp.bfloat16)
    return pl.pallas_call(
        _qkv_rope_kernel,
        grid=(H // 2,),
        in_specs=[
            pl.BlockSpec((S, D), lambda h: (0, 0)),
            pl.BlockSpec((D, 2 * DH), lambda h: (0, h)),
            pl.BlockSpec((D, 2 * DH), lambda h: (0, h)),
            pl.BlockSpec((D, 2 * DH), lambda h: (0, h)),
            pl.BlockSpec((S, DH), lambda h: (0, 0)),
            pl.BlockSpec((S, DH), lambda h: (0, 0)),
        ],
        out_specs=[
            pl.BlockSpec((2, NQT_PAD, DH), lambda h: (h, 0, 0)),
            pl.BlockSpec((2, NV, DH), lambda h: (h, 0, 0)),
            pl.BlockSpec((2, S, DH), lambda h: (h, 0, 0)),
            pl.BlockSpec((2, S, DH), lambda h: (h, 0, 0)),
            pl.BlockSpec((2, S, DH), lambda h: (h, 0, 0)),
        ],
        out_shape=[
            jax.ShapeDtypeStruct((H, NQT_PAD, DH), jnp.float32),
            jax.ShapeDtypeStruct((H, NV, DH), jnp.float32),
            outb, outb, outb,
        ],
    )(xb, wq, wk, wv, cos, sin)


def _scores_kernel(qt_ref, kv_ref, o_ref):
    qtb = qt_ref[0].astype(jnp.bfloat16)
    kvb = kv_ref[0].astype(jnp.bfloat16)
    sc = jax.lax.dot_general(
        qtb, kvb, (((1,), (1,)), ((), ())),
        preferred_element_type=jnp.float32)
    o_ref[0] = sc[:NQT]


def _scores(qt, kvis):
    return pl.pallas_call(
        _scores_kernel,
        grid=(H,),
        in_specs=[
            pl.BlockSpec((1, NQT_PAD, DH), lambda h: (h, 0, 0)),
            pl.BlockSpec((1, NV, DH), lambda h: (h, 0, 0)),
        ],
        out_specs=pl.BlockSpec((1, NQT, NV), lambda h: (h, 0, 0)),
        out_shape=jax.ShapeDtypeStruct((H, NQT, NV), jnp.float32),
    )(qt, kvis)


def _select_kernel(imp_ref, keep_ref):
    imp = imp_ref[...]
    a = jnp.broadcast_to(imp, (NV, NV))
    at = jnp.transpose(a)
    ii = jax.lax.broadcasted_iota(jnp.int32, (NV, NV), 0)
    jj = jax.lax.broadcasted_iota(jnp.int32, (NV, NV), 1)
    beats = (at > a) | ((at == a) & (ii < jj))
    rank = jnp.sum(beats.astype(jnp.int32), axis=0, keepdims=True)
    keep_ref[...] = (rank < KEEP_V).astype(jnp.int32)


def _select(imp):
    return pl.pallas_call(
        _select_kernel,
        out_shape=jax.ShapeDtypeStruct((1, NV), jnp.int32),
    )(imp)


_BQ = 512


def _attn_kernel(q_ref, k_ref, v_ref, pos_ref, o_ref, mask_ref, *, iblk, kext):
    h = pl.program_id(0)

    @pl.when(h == 0)
    def _():
        qpos = iblk * _BQ + jax.lax.broadcasted_iota(jnp.int32, (_BQ, 1), 0)
        allowed = pos_ref[0:1, :] <= qpos
        mask_ref[...] = jnp.where(allowed, 0.0, -jnp.inf)

    s = jax.lax.dot_general(
        q_ref[0], k_ref[0], (((1,), (1,)), ((), ())),
        preferred_element_type=jnp.float32)
    e = jnp.exp(s + mask_ref[...])
    den = jnp.sum(e, axis=1, keepdims=True)
    w = (e * (1.0 / den)).astype(jnp.bfloat16)
    o_ref[...] = jax.lax.dot(w, v_ref[0], preferred_element_type=jnp.float32)


def _attention(qb, kb, vb, posval):
    o_parts = []
    for i in range(S // _BQ):
        kext = (i + 1) * _BQ
        o_parts.append(pl.pallas_call(
            functools.partial(_attn_kernel, iblk=i, kext=kext),
            grid=(H,),
            in_specs=[
                pl.BlockSpec((1, _BQ, DH), lambda h, i=i: (h, i, 0)),
                pl.BlockSpec((1, kext, DH), lambda h: (h, 0, 0)),
                pl.BlockSpec((1, kext, DH), lambda h: (h, 0, 0)),
                pl.BlockSpec((8, kext), lambda h: (0, 0)),
            ],
            out_specs=pl.BlockSpec((_BQ, DH), lambda h: (0, h)),
            out_shape=jax.ShapeDtypeStruct((_BQ, H * DH), jnp.float32),
            scratch_shapes=[pltpu.VMEM((_BQ, kext), jnp.float32)],
        )(qb, kb, vb, posval))
    return jnp.concatenate(o_parts, axis=0)


_BSO = 512


def _outproj_kernel(o_ref, wo_ref, out_ref):
    ob = o_ref[...].astype(jnp.bfloat16)
    out_ref[...] = jax.lax.dot(ob, wo_ref[...].astype(jnp.bfloat16),
                               preferred_element_type=jnp.float32)


def _outproj(o, wo):
    return pl.pallas_call(
        _outproj_kernel,
        grid=(S // _BSO,),
        in_specs=[
            pl.BlockSpec((_BSO, H * DH), lambda i: (i, 0)),
            pl.BlockSpec((H * DH, D), lambda i: (0, 0)),
        ],
        out_specs=pl.BlockSpec((_BSO, D), lambda i: (i, 0)),
        out_shape=jax.ShapeDtypeStruct((S, D), jnp.float32),
    )(o, wo)


def kernel(hidden_states, Wq, Wk, Wv, Wo):
    x = hidden_states[0]
    cos, sin = _rope_tables()
    xb = x.astype(jnp.bfloat16)

    qt, kvis, qb, kb, vb = _qkv_rope(xb, Wq, Wk, Wv, cos, sin)

    scores = _scores(qt, kvis)[None] * _SCALE
    probs = jax.nn.softmax(scores.astype(jnp.float32), axis=-1)
    imp = probs.sum(axis=(0, 1, 2))
    keep = _select(imp[None])

    posidx = jnp.arange(S, dtype=jnp.int32)
    vis_pos = jnp.where(keep[0] == 1, posidx[VIS_START:VIS_END], _BIGPOS)
    posval = jnp.concatenate(
        [posidx[:VIS_START], vis_pos, posidx[VIS_END:]])
    posval = jnp.broadcast_to(posval[None, :], (8, S))

    o = _attention(qb, kb, vb, posval)
    out = _outproj(o, Wo)
    return out[None]
